# baseline (speedup 1.0000x reference)
import functools

import numpy as np
import jax
import jax.numpy as jnp
from jax import lax
from jax.experimental import pallas as pl
from jax.experimental.pallas import tpu as pltpu
from jax.experimental.pallas import tpu_sc as plsc

_EPS = 1e-10
_RSQRT_MAGIC = np.int32(0x5F3759DF)
_RECIP_MAGIC = np.int32(0x7EF311C3)


def _rsqrt16(x):
    y = plsc.bitcast(x, jnp.int32)
    y = _RSQRT_MAGIC - jnp.right_shift(y, 1)
    g = plsc.bitcast(y, jnp.float32)
    for _ in range(2):
        g = g * (1.5 - 0.5 * x * g * g)
    return g


def _recip16(x):
    y = _RECIP_MAGIC - plsc.bitcast(x, jnp.int32)
    g = plsc.bitcast(y, jnp.float32)
    for _ in range(2):
        g = g * (2.0 - x * g)
    return g


def _prologue_body(cand_ref, w1_ref, b1_ref, rdivt_ref, rmodt_ref, rmod_ref,
                   maski_ref, a_ref, c1b_ref):
    D = cand_ref.shape[1]
    f32 = jnp.float32
    w1c = w1_ref[0:D, :]
    w1b = w1_ref[D:2 * D, :]
    w1o = w1_ref[2 * D:, :]
    cand = cand_ref[...]
    c16 = jnp.dot(cand, rmod_ref[...], preferred_element_type=f32)
    repb = jnp.dot(rdivt_ref[...], c16, preferred_element_type=f32)
    e = repb * maski_ref[...]
    a_ref[...] = (jnp.dot(e, w1o, preferred_element_type=f32)
                  + jnp.dot(rmodt_ref[...], w1b, preferred_element_type=f32))
    c1b_ref[...] = jnp.dot(cand, w1c, preferred_element_type=f32) + b1_ref[...]


def _fold_weights(candidate_tensor, W1, b1):
    B, D = candidate_tensor.shape
    H = W1.shape[1]
    DD = D * D
    cols = jnp.arange(DD, dtype=jnp.int32)
    rows = jnp.arange(D, dtype=jnp.int32)
    r_div = (cols[None, :] // D == rows[:, None]).astype(jnp.float32)
    r_mod = (cols[None, :] % D == rows[:, None]).astype(jnp.float32)
    rr = jnp.arange(B * D, dtype=jnp.int32)
    maski = (rr[:, None] % D == cols[None, :] // D).astype(jnp.float32)
    return pl.pallas_call(
        _prologue_body,
        out_shape=[
            jax.ShapeDtypeStruct((B * D, H), jnp.float32),
            jax.ShapeDtypeStruct((B, H), jnp.float32),
        ],
    )(candidate_tensor, W1, b1.reshape(1, H), r_div.T, r_mod.T, r_mod, maski)


def _sc_body(beh_hbm, rid_hbm, a_hbm, c1b_hbm, c0_hbm, c1_hbm, w2_hbm,
             b2_hbm, out_hbm, beh_v, rid_v, a_v, c1b_v, c0_v, c1_v, w2_v,
             b2_v, h_v, acc_v):
    i32 = jnp.int32
    f32 = jnp.float32
    NC = 2
    TW = beh_v.shape[0] // 16
    NG = TW // 16

    wid = lax.axis_index("s") * NC + lax.axis_index("c")
    pltpu.sync_copy(beh_hbm.at[pl.ds(wid * (TW * 16), TW * 16)], beh_v)
    pltpu.sync_copy(rid_hbm.at[pl.ds(wid * TW, TW)], rid_v)
    pltpu.sync_copy(a_hbm, a_v)
    pltpu.sync_copy(c1b_hbm, c1b_v)
    pltpu.sync_copy(c0_hbm, c0_v)
    pltpu.sync_copy(c1_hbm, c1_v)
    pltpu.sync_copy(w2_hbm, w2_v)
    pltpu.sync_copy(b2_hbm, b2_v)

    for b in range(16):
        acc_v[pl.ds(b * 16, 16)] = jnp.zeros((16,), f32)

    lanes16 = lax.iota(i32, 16) * 16
    inv_h = 1.0 / 32.0

    def group(g, carry):
        off = g * 256
        r_vec = rid_v[pl.ds(g * 16, 16)]
        r32 = r_vec * 32
        r512 = r_vec * 512
        bt = [plsc.load_gather(beh_v, [lanes16 + (off + i)])
              for i in range(16)]

        def h_body(k, c):
            s, sq = c
            for j in range(2):
                hh = k * 2 + j
                chains = [plsc.load_gather(c1b_v, [r32 + hh])]
                chains += [
                    plsc.load_gather(a_v, [r512 + (i * 32 + hh)]) * bt[i]
                    for i in range(3)
                ]
                for i in range(3, 16):
                    c_id = (i - 3) & 3
                    chains[c_id] = chains[c_id] + plsc.load_gather(
                        a_v, [r512 + (i * 32 + hh)]) * bt[i]
                acc = (chains[0] + chains[1]) + (chains[2] + chains[3])
                h_v[pl.ds(hh * 16, 16)] = acc
                s = s + acc
                sq = sq + acc * acc
            return (s, sq)

        s, sq = lax.fori_loop(0, 16, h_body,
                              (jnp.zeros((16,), f32), jnp.zeros((16,), f32)))
        m = s * inv_h
        var = jnp.maximum(sq * inv_h - m * m, 0.0) + _EPS
        inv = _rsqrt16(var)
        invd = inv - _EPS * inv * inv

        def w_body(k, c):
            w0, w1 = c
            accs = []
            for j in range(2):
                hh = k * 2 + j
                hv = h_v[pl.ds(hh * 16, 16)]
                y = (hv - m) * invd
                p = _recip16(1.0 + jnp.exp(-y))
                c0r = c0_v[pl.ds(hh * 16, 16)]
                c1r = c1_v[pl.ds(hh * 16, 16)]
                w2r = w2_v[pl.ds(hh * 16, 16)]
                accs.append(hv * (c0r + c1r * p) * w2r)
            return (w0 + accs[0], w1 + accs[1])

        w0, w1 = lax.fori_loop(0, 16, w_body,
                               (b2_v[...], jnp.zeros((16,), f32)))
        wvec = w0 + w1
        r16 = r_vec * 16
        for i in range(16):
            plsc.addupdate_scatter(acc_v, [r16 + i], bt[i] * wvec)
        return carry

    lax.fori_loop(0, NG, group, 0)
    pltpu.sync_copy(acc_v, out_hbm.at[wid])


def kernel(candidate_tensor, behavior_flat_values, behavior_value_rowids, W1,
           b1, alpha, W2, b2):
    T, D = behavior_flat_values.shape
    B = candidate_tensor.shape[0]
    H = W1.shape[1]
    NW = 32
    TW = T // NW

    a_flat, c1b = _fold_weights(candidate_tensor, W1, b1)

    c0 = jnp.tile(alpha.reshape(H, 1), (1, 16)).reshape(-1)
    c1 = jnp.tile((1.0 - alpha).reshape(H, 1), (1, 16)).reshape(-1)
    w2r = jnp.tile(W2.reshape(H, 1), (1, 16)).reshape(-1)
    b2r = jnp.tile(b2.reshape(1, 1), (1, 16)).reshape(-1)

    mesh = plsc.VectorSubcoreMesh(core_axis_name="c", subcore_axis_name="s",
                                  num_cores=2, num_subcores=16)
    partials = functools.partial(
        pl.kernel,
        out_type=jax.ShapeDtypeStruct((NW, B * D), jnp.float32),
        mesh=mesh,
        compiler_params=pltpu.CompilerParams(needs_layout_passes=False),
        scratch_types=[
            pltpu.VMEM((TW * D,), jnp.float32),
            pltpu.VMEM((TW,), jnp.int32),
            pltpu.VMEM((B * D * H,), jnp.float32),
            pltpu.VMEM((B * H,), jnp.float32),
            pltpu.VMEM((H * 16,), jnp.float32),
            pltpu.VMEM((H * 16,), jnp.float32),
            pltpu.VMEM((H * 16,), jnp.float32),
            pltpu.VMEM((16,), jnp.float32),
            pltpu.VMEM((H * 16,), jnp.float32),
            pltpu.VMEM((B * D,), jnp.float32),
        ],
    )(_sc_body)(
        behavior_flat_values.reshape(-1),
        behavior_value_rowids,
        a_flat.reshape(-1),
        c1b.reshape(-1),
        c0, c1, w2r, b2r,
    )
    return partials.sum(axis=0).reshape(B, D)


# speedup vs baseline: 1.7517x; 1.7517x over previous
import functools

import numpy as np
import jax
import jax.numpy as jnp
from jax import lax
from jax.experimental import pallas as pl
from jax.experimental.pallas import tpu as pltpu
from jax.experimental.pallas import tpu_sc as plsc

_EPS = 1e-10
_RSQRT_MAGIC = np.int32(0x5F3759DF)
_RECIP_MAGIC = np.int32(0x7EF311C3)


def _rsqrt16(x):
    y = plsc.bitcast(x, jnp.int32)
    y = _RSQRT_MAGIC - jnp.right_shift(y, 1)
    g = plsc.bitcast(y, jnp.float32)
    for _ in range(2):
        g = g * (1.5 - 0.5 * x * g * g)
    return g


def _recip16(x):
    y = _RECIP_MAGIC - plsc.bitcast(x, jnp.int32)
    g = plsc.bitcast(y, jnp.float32)
    for _ in range(2):
        g = g * (2.0 - x * g)
    return g


def _prologue_body(cand_ref, w1_ref, b1_ref, rdivt_ref, rmodt_ref, rmod_ref,
                   maski_ref, a_ref, c1b_ref):
    D = cand_ref.shape[1]
    f32 = jnp.float32
    w1c = w1_ref[0:D, :]
    w1b = w1_ref[D:2 * D, :]
    w1o = w1_ref[2 * D:, :]
    cand = cand_ref[...]
    c16 = jnp.dot(cand, rmod_ref[...], preferred_element_type=f32)
    repb = jnp.dot(rdivt_ref[...], c16, preferred_element_type=f32)
    e = repb * maski_ref[...]
    a_ref[...] = (jnp.dot(e, w1o, preferred_element_type=f32)
                  + jnp.dot(rmodt_ref[...], w1b, preferred_element_type=f32))
    c1b_ref[...] = jnp.dot(cand, w1c, preferred_element_type=f32) + b1_ref[...]


def _fold_weights(candidate_tensor, W1, b1):
    B, D = candidate_tensor.shape
    H = W1.shape[1]
    DD = D * D
    cols = jnp.arange(DD, dtype=jnp.int32)
    rows = jnp.arange(D, dtype=jnp.int32)
    r_div = (cols[None, :] // D == rows[:, None]).astype(jnp.float32)
    r_mod = (cols[None, :] % D == rows[:, None]).astype(jnp.float32)
    rr = jnp.arange(B * D, dtype=jnp.int32)
    maski = (rr[:, None] % D == cols[None, :] // D).astype(jnp.float32)
    return pl.pallas_call(
        _prologue_body,
        out_shape=[
            jax.ShapeDtypeStruct((B * D, H), jnp.float32),
            jax.ShapeDtypeStruct((B, H), jnp.float32),
        ],
    )(candidate_tensor, W1, b1.reshape(1, H), r_div.T, r_mod.T, r_mod, maski)


def _tc_body(cand_ref, beh_ref, rid_ref, w1_ref, b1_ref, alpha_ref, w2_ref,
             b2_ref, rdiv_ref, rmod_ref, out_ref):
    i = pl.program_id(0)
    TB, D = beh_ref.shape
    B = cand_ref.shape[0]
    H = w1_ref.shape[1]
    f32 = jnp.float32

    beh = beh_ref[...]
    r = rid_ref[...]
    bidx = lax.broadcasted_iota(jnp.int32, (TB, B), 1)
    P = (r == bidx).astype(f32)

    w1c = w1_ref[0:D, :]
    w1b = w1_ref[D:2 * D, :]
    w1o = w1_ref[2 * D:, :]

    cand = cand_ref[...]
    cand_tiled = jnp.dot(cand, rmod_ref[...], preferred_element_type=f32)
    c1b = jnp.dot(cand, w1c, preferred_element_type=f32) + b1_ref[...]

    xb = jnp.dot(beh, rdiv_ref[...], preferred_element_type=f32)
    xc = jnp.dot(P, cand_tiled, preferred_element_type=f32)
    outer = xb * xc
    h = (jnp.dot(outer, w1o, preferred_element_type=f32)
         + jnp.dot(beh, w1b, preferred_element_type=f32)
         + jnp.dot(P, c1b, preferred_element_type=f32))

    v_mean = jnp.full((H, H), 1.0 / H, f32)
    mean = jnp.dot(h, v_mean, preferred_element_type=f32)
    msq = jnp.dot(h * h, v_mean, preferred_element_type=f32)
    var = jnp.maximum(msq - mean * mean, 0.0) + _EPS
    inv2 = 0.5 * lax.rsqrt(var)
    th = jnp.tanh((h - mean) * inv2)
    alpha_v = alpha_ref[...]
    c0 = 0.5 * (1.0 + alpha_v)
    c1 = 0.5 * (1.0 - alpha_v)
    hd = h * (c0 + c1 * th)

    w = jnp.dot(hd, w2_ref[...], preferred_element_type=f32) + b2_ref[...]
    weighted = beh * w
    partial = lax.dot_general(P, weighted, (((0,), (0,)), ((), ())),
                              preferred_element_type=f32)

    @pl.when(i == 0)
    def _init():
        out_ref[...] = jnp.zeros_like(out_ref)

    out_ref[...] += partial


def _tc_part(candidate_tensor, behavior, rowids, W1, b1, alpha, W2, b2):
    T, D = behavior.shape
    B = candidate_tensor.shape[0]
    H = W1.shape[1]
    DD = D * D
    TB = 4096
    grid = T // TB

    rowids2 = rowids.reshape(T, 1)
    b1r = b1.reshape(1, H)
    alphar = alpha.reshape(1, H)
    w2rep = jnp.tile(W2.reshape(H, 1), (1, D))
    b2r = b2.reshape(1, 1)
    cols = jnp.arange(DD, dtype=jnp.int32)
    rows = jnp.arange(D, dtype=jnp.int32)
    r_div = (cols[None, :] // D == rows[:, None]).astype(jnp.float32)
    r_mod = (cols[None, :] % D == rows[:, None]).astype(jnp.float32)

    return pl.pallas_call(
        _tc_body,
        grid=(grid,),
        in_specs=[
            pl.BlockSpec((B, D), lambda i: (0, 0)),
            pl.BlockSpec((TB, D), lambda i: (i, 0)),
            pl.BlockSpec((TB, 1), lambda i: (i, 0)),
            pl.BlockSpec((D + D + DD, H), lambda i: (0, 0)),
            pl.BlockSpec((1, H), lambda i: (0, 0)),
            pl.BlockSpec((1, H), lambda i: (0, 0)),
            pl.BlockSpec((H, D), lambda i: (0, 0)),
            pl.BlockSpec((1, 1), lambda i: (0, 0)),
            pl.BlockSpec((D, DD), lambda i: (0, 0)),
            pl.BlockSpec((D, DD), lambda i: (0, 0)),
        ],
        out_specs=pl.BlockSpec((B, D), lambda i: (0, 0)),
        out_shape=jax.ShapeDtypeStruct((B, D), jnp.float32),
    )(candidate_tensor, behavior, rowids2, W1, b1r, alphar, w2rep, b2r,
      r_div, r_mod)


def _sc_body(beh_hbm, rid_hbm, a_hbm, c1b_hbm, c0_hbm, c1_hbm, w2_hbm,
             b2_hbm, out_hbm, beh_v, rid_v, a_v, c1b_v, c0_v, c1_v, w2_v,
             b2_v, h_v, acc_v):
    i32 = jnp.int32
    f32 = jnp.float32
    NC = 2
    TW = beh_v.shape[0] // 16
    NG = TW // 16

    wid = lax.axis_index("s") * NC + lax.axis_index("c")
    pltpu.sync_copy(beh_hbm.at[pl.ds(wid * (TW * 16), TW * 16)], beh_v)
    pltpu.sync_copy(rid_hbm.at[pl.ds(wid * TW, TW)], rid_v)
    pltpu.sync_copy(a_hbm, a_v)
    pltpu.sync_copy(c1b_hbm, c1b_v)
    pltpu.sync_copy(c0_hbm, c0_v)
    pltpu.sync_copy(c1_hbm, c1_v)
    pltpu.sync_copy(w2_hbm, w2_v)
    pltpu.sync_copy(b2_hbm, b2_v)

    for b in range(16):
        acc_v[pl.ds(b * 16, 16)] = jnp.zeros((16,), f32)

    lanes16 = lax.iota(i32, 16) * 16
    inv_h = 1.0 / 32.0

    def group(g, carry):
        off = g * 256
        r_vec = rid_v[pl.ds(g * 16, 16)]
        r32 = r_vec * 32
        r512 = r_vec * 512
        bt = [plsc.load_gather(beh_v, [lanes16 + (off + i)])
              for i in range(16)]

        def h_body(k, c):
            s, sq = c
            for j in range(2):
                hh = k * 2 + j
                chains = [plsc.load_gather(c1b_v, [r32 + hh])]
                chains += [
                    plsc.load_gather(a_v, [r512 + (i * 32 + hh)]) * bt[i]
                    for i in range(3)
                ]
                for i in range(3, 16):
                    c_id = (i - 3) & 3
                    chains[c_id] = chains[c_id] + plsc.load_gather(
                        a_v, [r512 + (i * 32 + hh)]) * bt[i]
                acc = (chains[0] + chains[1]) + (chains[2] + chains[3])
                h_v[pl.ds(hh * 16, 16)] = acc
                s = s + acc
                sq = sq + acc * acc
            return (s, sq)

        s, sq = lax.fori_loop(0, 16, h_body,
                              (jnp.zeros((16,), f32), jnp.zeros((16,), f32)))
        m = s * inv_h
        var = jnp.maximum(sq * inv_h - m * m, 0.0) + _EPS
        inv = _rsqrt16(var)
        invd = inv - _EPS * inv * inv

        def w_body(k, c):
            w0, w1 = c
            accs = []
            for j in range(2):
                hh = k * 2 + j
                hv = h_v[pl.ds(hh * 16, 16)]
                y = (hv - m) * invd
                p = _recip16(1.0 + jnp.exp(-y))
                c0r = c0_v[pl.ds(hh * 16, 16)]
                c1r = c1_v[pl.ds(hh * 16, 16)]
                w2r = w2_v[pl.ds(hh * 16, 16)]
                accs.append(hv * (c0r + c1r * p) * w2r)
            return (w0 + accs[0], w1 + accs[1])

        w0, w1 = lax.fori_loop(0, 16, w_body,
                               (b2_v[...], jnp.zeros((16,), f32)))
        wvec = w0 + w1
        r16 = r_vec * 16
        for i in range(16):
            plsc.addupdate_scatter(acc_v, [r16 + i], bt[i] * wvec)
        return carry

    lax.fori_loop(0, NG, group, 0)
    pltpu.sync_copy(acc_v, out_hbm.at[wid])


def kernel(candidate_tensor, behavior_flat_values, behavior_value_rowids, W1,
           b1, alpha, W2, b2):
    T, D = behavior_flat_values.shape
    B = candidate_tensor.shape[0]
    H = W1.shape[1]
    NW = 32
    T_SC = 8192
    TW = T_SC // NW

    a_flat, c1b = _fold_weights(candidate_tensor, W1, b1)

    c0 = jnp.tile(alpha.reshape(H, 1), (1, 16)).reshape(-1)
    c1 = jnp.tile((1.0 - alpha).reshape(H, 1), (1, 16)).reshape(-1)
    w2r = jnp.tile(W2.reshape(H, 1), (1, 16)).reshape(-1)
    b2r = jnp.tile(b2.reshape(1, 1), (1, 16)).reshape(-1)

    mesh = plsc.VectorSubcoreMesh(core_axis_name="c", subcore_axis_name="s",
                                  num_cores=2, num_subcores=16)
    partials = functools.partial(
        pl.kernel,
        out_type=jax.ShapeDtypeStruct((NW, B * D), jnp.float32),
        mesh=mesh,
        compiler_params=pltpu.CompilerParams(needs_layout_passes=False),
        scratch_types=[
            pltpu.VMEM((TW * D,), jnp.float32),
            pltpu.VMEM((TW,), jnp.int32),
            pltpu.VMEM((B * D * H,), jnp.float32),
            pltpu.VMEM((B * H,), jnp.float32),
            pltpu.VMEM((H * 16,), jnp.float32),
            pltpu.VMEM((H * 16,), jnp.float32),
            pltpu.VMEM((H * 16,), jnp.float32),
            pltpu.VMEM((16,), jnp.float32),
            pltpu.VMEM((H * 16,), jnp.float32),
            pltpu.VMEM((B * D,), jnp.float32),
        ],
    )(_sc_body)(
        behavior_flat_values[:T_SC].reshape(-1),
        behavior_value_rowids[:T_SC],
        a_flat.reshape(-1),
        c1b.reshape(-1),
        c0, c1, w2r, b2r,
    )
    tc_out = _tc_part(candidate_tensor, behavior_flat_values[T_SC:],
                      behavior_value_rowids[T_SC:], W1, b1, alpha, W2, b2)
    return partials.sum(axis=0).reshape(B, D) + tc_out
